# Initial kernel scaffold; baseline (speedup 1.0000x reference)
import jax
import jax.numpy as jnp
from jax.experimental import pallas as pl


def kernel(q, k, v, Wq, bq, Wk, bk, Wv, bv, Wo, bo, g1, be1, g2, be2, Wg, bg, W1, b1, W2, b2):
    raise NotImplementedError("write your pallas kernel here")


# speedup vs baseline: 1.6266x; 1.6266x over previous
import functools

import jax
import jax.numpy as jnp
from jax import lax
from jax.experimental import pallas as pl
from jax.experimental.pallas import tpu as pltpu
from jax.experimental.pallas import tpu_sc as plsc

S = 2048
D = 768
H = 12
DH = D // H
E = 8
F = 3072
BK = 256
NP = 2 * S
NB = NP // BK + E
NPAD = NP + E * BK
NW = 32
TPW = S // NW
_NEG = -1e30


def _qkv_body(q_ref, k_ref, v_ref, wq_ref, bq_ref, wk_ref, bk_ref,
              wv_ref, bv_ref, oq_ref, ok_ref, ov_ref):
    oq_ref[...] = jnp.dot(q_ref[...], wq_ref[...],
                          preferred_element_type=jnp.float32) + bq_ref[...]
    ok_ref[...] = jnp.dot(k_ref[...], wk_ref[...],
                          preferred_element_type=jnp.float32) + bk_ref[...]
    ov_ref[...] = jnp.dot(v_ref[...], wv_ref[...],
                          preferred_element_type=jnp.float32) + bv_ref[...]


def _qkv(q, k, v, wq, bq, wk, bk, wv, bv):
    shp = jax.ShapeDtypeStruct((S, D), jnp.float32)
    return pl.pallas_call(_qkv_body, out_shape=(shp, shp, shp))(
        q, k, v, wq, bq, wk, bk, wv, bv)


def _attn_body(q_ref, k_ref, v_ref, o_ref):
    scale = jnp.float32(1.0 / (DH ** 0.5))
    for j in range(2):
        qh = q_ref[:, j * DH:(j + 1) * DH] * scale
        kh = k_ref[:, j * DH:(j + 1) * DH]
        vh = v_ref[:, j * DH:(j + 1) * DH]
        s = lax.dot_general(qh, kh, (((1,), (1,)), ((), ())),
                            preferred_element_type=jnp.float32)
        m = jnp.max(s, axis=-1, keepdims=True)
        p = jnp.exp(s - m)
        den = jnp.sum(p, axis=-1, keepdims=True)
        o = jnp.dot(p, vh, preferred_element_type=jnp.float32)
        o_ref[:, j * DH:(j + 1) * DH] = o / den


def _attention(qp, kp, vp):
    spec = pl.BlockSpec((S, 2 * DH), lambda h: (0, h))
    return pl.pallas_call(
        _attn_body,
        grid=(H // 2,),
        in_specs=[spec, spec, spec],
        out_specs=spec,
        out_shape=jax.ShapeDtypeStruct((S, D), jnp.float32),
    )(qp, kp, vp)


def _post_body(ao_ref, q_ref, wo_ref, bo_ref, g1_ref, be1_ref, wg_ref,
               bg_ref, x_ref, dest_ref, w1_ref, w2_ref, pb_ref):
    proj = jnp.dot(ao_ref[...], wo_ref[...],
                   preferred_element_type=jnp.float32) + bo_ref[...]
    y = q_ref[...] + proj
    mu = jnp.mean(y, axis=-1, keepdims=True)
    var = jnp.mean((y - mu) ** 2, axis=-1, keepdims=True)
    x = (y - mu) * lax.rsqrt(var + 1e-5) * g1_ref[...] + be1_ref[...]
    x_ref[...] = x

    logits = jnp.dot(x, wg_ref[...],
                     preferred_element_type=jnp.float32) + bg_ref[...]
    ie = lax.broadcasted_iota(jnp.int32, (S, E), 1).astype(jnp.float32)
    m1 = jnp.max(logits, axis=-1, keepdims=True)
    i1 = jnp.min(jnp.where(logits == m1, ie, jnp.float32(E)),
                 axis=-1, keepdims=True)
    l2 = jnp.where(ie == i1, _NEG, logits)
    m2 = jnp.max(l2, axis=-1, keepdims=True)
    i2 = jnp.min(jnp.where(l2 == m2, ie, jnp.float32(E)),
                 axis=-1, keepdims=True)
    w1_ref[...] = 1.0 / (1.0 + jnp.exp(m2 - m1))
    w2_ref[...] = 1.0 - 1.0 / (1.0 + jnp.exp(m2 - m1))

    oh1 = (ie == i1).astype(jnp.float32)
    oh2 = (ie == i2).astype(jnp.float32)
    m_pairs = jnp.concatenate([oh1, oh2], axis=0)

    C = 128
    ir = lax.broadcasted_iota(jnp.int32, (C, C), 0)
    ic = lax.broadcasted_iota(jnp.int32, (C, C), 1)
    ltri = (ic < ir).astype(jnp.float32)
    off = jnp.zeros((1, E), jnp.float32)
    ranks = []
    for c in range(NP // C):
        blk = m_pairs[c * C:(c + 1) * C]
        ranks.append(jnp.dot(ltri, blk,
                             preferred_element_type=jnp.float32) + off)
        off = off + jnp.sum(blk, axis=0, keepdims=True)
    rank = jnp.concatenate(ranks, axis=0)

    cnt = off
    cpad = jnp.ceil(cnt / BK) * BK
    utri = (lax.broadcasted_iota(jnp.int32, (E, E), 0) <
            lax.broadcasted_iota(jnp.int32, (E, E), 1)).astype(jnp.float32)
    opad = jnp.dot(cpad, utri, preferred_element_type=jnp.float32)
    dest = jnp.sum(m_pairs * (rank + opad), axis=-1, keepdims=True)
    dest_ref[...] = dest.astype(jnp.int32)

    ends = opad + cpad
    row = lax.broadcasted_iota(jnp.int32, (NB + 1, 1), 0).astype(jnp.float32)
    ebm = jnp.sum((row * BK >= ends).astype(jnp.float32),
                  axis=-1, keepdims=True)
    eb = jnp.clip(ebm, 0.0, float(E - 1))
    nactive = ends[:, E - 1:E] / BK
    pb = jnp.where(row < NB, eb, nactive)
    pb_ref[...] = pb.astype(jnp.int32)


def _post_attn(ao, q2d, wo, bo, g1, be1, wg, bg):
    return pl.pallas_call(
        _post_body,
        out_shape=(
            jax.ShapeDtypeStruct((S, D), jnp.float32),
            jax.ShapeDtypeStruct((NP, 1), jnp.int32),
            jax.ShapeDtypeStruct((S, 1), jnp.float32),
            jax.ShapeDtypeStruct((S, 1), jnp.float32),
            jax.ShapeDtypeStruct((NB + 1, 1), jnp.int32),
        ),
    )(ao, q2d, wo, bo, g1, be1, wg, bg)


@functools.lru_cache(maxsize=None)
def _sc_kernels():
    mesh = plsc.VectorSubcoreMesh(core_axis_name="c", subcore_axis_name="s")

    @functools.partial(
        pl.kernel,
        out_type=jax.ShapeDtypeStruct((NPAD, D), jnp.float32),
        mesh=mesh,
        scratch_types=[
            pltpu.VMEM((TPW, D), jnp.float32),
            pltpu.VMEM((TPW,), jnp.int32),
            pltpu.VMEM((TPW,), jnp.int32),
            pltpu.SemaphoreType.DMA,
        ],
    )
    def sc_dispatch(x_hbm, idx1_hbm, idx2_hbm, xs_hbm, rows_v, i1_v, i2_v,
                    sem):
        wid = lax.axis_index("s") * 2 + lax.axis_index("c")
        base = wid * TPW
        pltpu.sync_copy(x_hbm.at[pl.ds(base, TPW)], rows_v)
        pltpu.sync_copy(idx1_hbm.at[wid], i1_v)
        pltpu.sync_copy(idx2_hbm.at[wid], i2_v)
        pltpu.async_copy(rows_v, xs_hbm.at[i1_v], sem).wait()
        pltpu.async_copy(rows_v, xs_hbm.at[i2_v], sem).wait()

    @functools.partial(
        pl.kernel,
        out_type=(jax.ShapeDtypeStruct((S, D), jnp.float32),
                  jax.ShapeDtypeStruct((S, D), jnp.float32)),
        mesh=mesh,
        scratch_types=[
            pltpu.VMEM((TPW, D), jnp.float32),
            pltpu.VMEM((TPW,), jnp.int32),
            pltpu.SemaphoreType.DMA,
        ],
    )
    def sc_combine(y_hbm, idx1_hbm, idx2_hbm, y1_hbm, y2_hbm, rows_v, i_v,
                   sem):
        wid = lax.axis_index("s") * 2 + lax.axis_index("c")
        base = wid * TPW
        pltpu.sync_copy(idx1_hbm.at[wid], i_v)
        pltpu.async_copy(y_hbm.at[i_v], rows_v, sem).wait()
        pltpu.sync_copy(rows_v, y1_hbm.at[pl.ds(base, TPW)])
        pltpu.sync_copy(idx2_hbm.at[wid], i_v)
        pltpu.async_copy(y_hbm.at[i_v], rows_v, sem).wait()
        pltpu.sync_copy(rows_v, y2_hbm.at[pl.ds(base, TPW)])

    return sc_dispatch, sc_combine


def _dispatch(x2d, d1, d2):
    return _sc_kernels()[0](x2d, d1, d2)


def _combine(y, d1, d2):
    return _sc_kernels()[1](y, d1, d2)


def _ffn_body(pb_ref, xs_ref, w1_ref, b1_ref, w2_ref, b2_ref, y_ref):
    b = pl.program_id(0)

    @pl.when(b < pb_ref[NB])
    def _():
        h = jnp.dot(xs_ref[...], w1_ref[0],
                    preferred_element_type=jnp.float32) + b1_ref[0]
        h = jax.nn.gelu(h)
        y_ref[...] = jnp.dot(h, w2_ref[0],
                             preferred_element_type=jnp.float32) + b2_ref[0]


def _ffn(pb, xs, w1, b1, w2, b2):
    grid_spec = pltpu.PrefetchScalarGridSpec(
        num_scalar_prefetch=1,
        grid=(NB,),
        in_specs=[
            pl.BlockSpec((BK, D), lambda b, pb: (b, 0)),
            pl.BlockSpec((1, D, F), lambda b, pb: (pb[b], 0, 0)),
            pl.BlockSpec((1, 1, F), lambda b, pb: (pb[b], 0, 0)),
            pl.BlockSpec((1, F, D), lambda b, pb: (pb[b], 0, 0)),
            pl.BlockSpec((1, 1, D), lambda b, pb: (pb[b], 0, 0)),
        ],
        out_specs=pl.BlockSpec((BK, D), lambda b, pb: (b, 0)),
    )
    return pl.pallas_call(
        _ffn_body,
        grid_spec=grid_spec,
        out_shape=jax.ShapeDtypeStruct((NPAD, D), jnp.float32),
    )(pb, xs, w1, b1, w2, b2)


def _final_body(x_ref, y1_ref, y2_ref, w1_ref, w2_ref, g2_ref, be2_ref,
                o_ref):
    y = x_ref[...] + w1_ref[...] * y1_ref[...] + w2_ref[...] * y2_ref[...]
    mu = jnp.mean(y, axis=-1, keepdims=True)
    var = jnp.mean((y - mu) ** 2, axis=-1, keepdims=True)
    o_ref[...] = (y - mu) * lax.rsqrt(var + 1e-5) * g2_ref[...] + be2_ref[...]


def _final(x, y1, y2, w1, w2, g2, be2):
    return pl.pallas_call(
        _final_body,
        out_shape=jax.ShapeDtypeStruct((S, D), jnp.float32),
    )(x, y1, y2, w1, w2, g2, be2)


def kernel(q, k, v, Wq, bq, Wk, bk, Wv, bv, Wo, bo, g1, be1, g2, be2,
           Wg, bg, W1, b1, W2, b2):
    q2d = q.reshape(S, D)
    k2d = k.reshape(S, D)
    v2d = v.reshape(S, D)
    qp, kp, vp = _qkv(q2d, k2d, v2d, Wq, bq.reshape(1, D), Wk,
                      bk.reshape(1, D), Wv, bv.reshape(1, D))
    ao = _attention(qp, kp, vp)
    x, dest, w1, w2, pb = _post_attn(
        ao, q2d, Wo, bo.reshape(1, D), g1.reshape(1, D), be1.reshape(1, D),
        Wg, bg.reshape(1, E))
    d1 = dest[:S, 0].reshape(NW, TPW)
    d2 = dest[S:, 0].reshape(NW, TPW)
    xs = _dispatch(x, d1, d2)
    y = _ffn(pb.reshape(NB + 1), xs, W1, b1.reshape(E, 1, F),
             W2, b2.reshape(E, 1, D))
    y1, y2 = _combine(y, d1, d2)
    out = _final(x, y1, y2, w1, w2, g2.reshape(1, D), be2.reshape(1, D))
    return out.reshape(q.shape)
